# baseline (device time: 87722 ns/iter reference)
import jax
import jax.numpy as jnp
from jax import lax
from jax.experimental import pallas as pl
from jax.experimental.pallas import tpu as pltpu

N_DEV = 4
M = 1024
D = 1024
CH = M // N_DEV


def kernel(partial, resid, gamma):
    x = partial.reshape(M, D)
    g = gamma.reshape(1, D)

    def body(x_ref, resid_ref, g_ref, out_ref, rs_bufs, send_sems, recv_sems):
        p = lax.axis_index("i")
        left = (p - 1) % N_DEV
        right = (p + 1) % N_DEV

        barrier_sem = pltpu.get_barrier_semaphore()
        for nbr in (left, right):
            pl.semaphore_signal(
                barrier_sem, inc=1,
                device_id=(nbr,), device_id_type=pl.DeviceIdType.MESH,
            )
        pl.semaphore_wait(barrier_sem, 2)

        rdma0 = pltpu.make_async_remote_copy(
            src_ref=x_ref.at[pl.ds((p % N_DEV) * CH, CH), :],
            dst_ref=rs_bufs.at[0],
            send_sem=send_sems.at[0],
            recv_sem=recv_sems.at[0],
            device_id=(right,),
            device_id_type=pl.DeviceIdType.MESH,
        )
        rdma0.start()
        rdma0.wait()
        c1 = (p - 1) % N_DEV
        rs_bufs[0, :, :] = rs_bufs[0, :, :] + x_ref[pl.ds(c1 * CH, CH), :]

        rdma1 = pltpu.make_async_remote_copy(
            src_ref=rs_bufs.at[0],
            dst_ref=rs_bufs.at[1],
            send_sem=send_sems.at[1],
            recv_sem=recv_sems.at[1],
            device_id=(right,),
            device_id_type=pl.DeviceIdType.MESH,
        )
        rdma1.start()
        rdma1.wait()
        c2 = (p - 2) % N_DEV
        rs_bufs[1, :, :] = rs_bufs[1, :, :] + x_ref[pl.ds(c2 * CH, CH), :]

        rdma2 = pltpu.make_async_remote_copy(
            src_ref=rs_bufs.at[1],
            dst_ref=rs_bufs.at[2],
            send_sem=send_sems.at[2],
            recv_sem=recv_sems.at[2],
            device_id=(right,),
            device_id_type=pl.DeviceIdType.MESH,
        )
        rdma2.start()
        rdma2.wait()
        r = (p + 1) % N_DEV
        rows = pl.ds(r * CH, CH)
        acc = rs_bufs[2, :, :] + x_ref[rows, :]

        y = acc + resid_ref[rows, :]
        ms = jnp.mean(y * y, axis=1, keepdims=True)
        inv = lax.rsqrt(ms + 1e-6)
        out_ref[rows, :] = y * inv * g_ref[0, :]

        for s in range(N_DEV - 1):
            c_send = (r - s) % N_DEV
            srows = pl.ds(c_send * CH, CH)
            rdma = pltpu.make_async_remote_copy(
                src_ref=out_ref.at[srows, :],
                dst_ref=out_ref.at[srows, :],
                send_sem=send_sems.at[3 + s],
                recv_sem=recv_sems.at[3 + s],
                device_id=(right,),
                device_id_type=pl.DeviceIdType.MESH,
            )
            rdma.start()
            rdma.wait()

    return pl.pallas_call(
        body,
        out_shape=jax.ShapeDtypeStruct((M, D), jnp.float32),
        in_specs=[
            pl.BlockSpec(memory_space=pltpu.VMEM),
            pl.BlockSpec(memory_space=pltpu.VMEM),
            pl.BlockSpec(memory_space=pltpu.VMEM),
        ],
        out_specs=pl.BlockSpec(memory_space=pltpu.VMEM),
        scratch_shapes=[
            pltpu.VMEM((3, CH, D), jnp.float32),
            pltpu.SemaphoreType.DMA((6,)),
            pltpu.SemaphoreType.DMA((6,)),
        ],
        compiler_params=pltpu.CompilerParams(collective_id=0),
    )(x, resid, g)


# device time: 50705 ns/iter; 1.7300x vs baseline; 1.7300x over previous
import jax
import jax.numpy as jnp
from jax import lax
from jax.experimental import pallas as pl
from jax.experimental.pallas import tpu as pltpu

N_DEV = 4
M = 1024
D = 1024
HALF = M // 2
QUART = HALF // 2
EIGHT = HALF // 4


def kernel(partial, resid, gamma):
    x = partial.reshape(M, D)
    g = gamma.reshape(1, D)

    def body(x_ref, resid_ref, g_ref, out_ref, r1, r2, r3, r4,
             send_sems, recv_sems):
        p = lax.axis_index("i")
        q1 = p + 1 - 2 * (p % 2)
        q2 = 3 - p

        def xchg(idx, src, dst, partner):
            rdma = pltpu.make_async_remote_copy(
                src_ref=src, dst_ref=dst,
                send_sem=send_sems.at[idx], recv_sem=recv_sems.at[idx],
                device_id=(partner,), device_id_type=pl.DeviceIdType.MESH,
            )
            rdma.start()
            return rdma

        a1 = jnp.where((p == 0) | (p == 3), 0, QUART)
        c1 = jnp.where(p <= 1, 0, EIGHT)
        a2 = jnp.where(p <= 1, 0, QUART)
        c2 = jnp.where((p == 0) | (p == 2), 0, EIGHT)

        barrier_sem = pltpu.get_barrier_semaphore()
        for nbr in (q1, q2):
            pl.semaphore_signal(
                barrier_sem, inc=1,
                device_id=(nbr,), device_id_type=pl.DeviceIdType.MESH,
            )
        pl.semaphore_wait(barrier_sem, 2)

        s1a = xchg(0, x_ref.at[pl.ds((QUART - a1), QUART), :], r1, q1)
        s1b = xchg(1, x_ref.at[pl.ds(HALF + (QUART - a2), QUART), :], r2, q2)

        s1a.wait()
        r1[:, :] = r1[:, :] + x_ref[pl.ds(a1, QUART), :]
        s2a = xchg(2, r1.at[pl.ds(EIGHT - c1, EIGHT), :], r3, q2)

        s1b.wait()
        r2[:, :] = r2[:, :] + x_ref[pl.ds(HALF + a2, QUART), :]
        s2b = xchg(3, r2.at[pl.ds(EIGHT - c2, EIGHT), :], r4, q1)

        o1 = a1 + c1
        o2 = HALF + a2 + c2

        s2a.wait()
        y1 = r3[:, :] + r1[pl.ds(c1, EIGHT), :] + resid_ref[pl.ds(o1, EIGHT), :]
        inv1 = lax.rsqrt(jnp.mean(y1 * y1, axis=1, keepdims=True) + 1e-6)
        out_ref[pl.ds(o1, EIGHT), :] = y1 * inv1 * g_ref[0, :]
        s3a = xchg(4, out_ref.at[pl.ds(o1, EIGHT), :],
                   out_ref.at[pl.ds(o1, EIGHT), :], q2)

        s2b.wait()
        y2 = r4[:, :] + r2[pl.ds(c2, EIGHT), :] + resid_ref[pl.ds(o2, EIGHT), :]
        inv2 = lax.rsqrt(jnp.mean(y2 * y2, axis=1, keepdims=True) + 1e-6)
        out_ref[pl.ds(o2, EIGHT), :] = y2 * inv2 * g_ref[0, :]
        s3b = xchg(5, out_ref.at[pl.ds(o2, EIGHT), :],
                   out_ref.at[pl.ds(o2, EIGHT), :], q1)

        s3a.wait()
        s4a = xchg(6, out_ref.at[pl.ds(a1, QUART), :],
                   out_ref.at[pl.ds(a1, QUART), :], q1)
        s3b.wait()
        s4b = xchg(7, out_ref.at[pl.ds(HALF + a2, QUART), :],
                   out_ref.at[pl.ds(HALF + a2, QUART), :], q2)

        s4a.wait()
        s4b.wait()

    return pl.pallas_call(
        body,
        out_shape=jax.ShapeDtypeStruct((M, D), jnp.float32),
        in_specs=[
            pl.BlockSpec(memory_space=pltpu.VMEM),
            pl.BlockSpec(memory_space=pltpu.VMEM),
            pl.BlockSpec(memory_space=pltpu.VMEM),
        ],
        out_specs=pl.BlockSpec(memory_space=pltpu.VMEM),
        scratch_shapes=[
            pltpu.VMEM((QUART, D), jnp.float32),
            pltpu.VMEM((QUART, D), jnp.float32),
            pltpu.VMEM((EIGHT, D), jnp.float32),
            pltpu.VMEM((EIGHT, D), jnp.float32),
            pltpu.SemaphoreType.DMA((8,)),
            pltpu.SemaphoreType.DMA((8,)),
        ],
        compiler_params=pltpu.CompilerParams(collective_id=0),
    )(x, resid, g)


# device time: 50650 ns/iter; 1.7319x vs baseline; 1.0011x over previous
import jax
import jax.numpy as jnp
from jax import lax
from jax.experimental import pallas as pl
from jax.experimental.pallas import tpu as pltpu

N_DEV = 4
M = 1024
D = 1024
HALF = M // 2
QUART = HALF // 2
EIGHT = HALF // 4


def kernel(partial, resid, gamma):
    x = partial.reshape(M, D)
    g = gamma.reshape(1, D)

    def body(x_ref, resid_ref, g_ref, out_ref, r1, r2, r3, r4, t1, t2,
             send_sems, recv_sems):
        p = lax.axis_index("i")
        q1 = p + 1 - 2 * (p % 2)
        q2 = 3 - p

        def xchg(idx, src, dst, partner):
            rdma = pltpu.make_async_remote_copy(
                src_ref=src, dst_ref=dst,
                send_sem=send_sems.at[idx], recv_sem=recv_sems.at[idx],
                device_id=(partner,), device_id_type=pl.DeviceIdType.MESH,
            )
            rdma.start()
            return rdma

        a1 = jnp.where((p == 0) | (p == 3), 0, QUART)
        c1 = jnp.where(p <= 1, 0, EIGHT)
        a2 = jnp.where(p <= 1, 0, QUART)
        c2 = jnp.where((p == 0) | (p == 2), 0, EIGHT)

        barrier_sem = pltpu.get_barrier_semaphore()
        for nbr in (q1, q2):
            pl.semaphore_signal(
                barrier_sem, inc=1,
                device_id=(nbr,), device_id_type=pl.DeviceIdType.MESH,
            )
        pl.semaphore_wait(barrier_sem, 2)

        o1 = a1 + c1
        o2 = HALF + a2 + c2
        f1 = EIGHT - c1
        f2 = EIGHT - c2

        s1a = xchg(0, x_ref.at[pl.ds((QUART - a1), QUART), :], r1, q1)
        s1b = xchg(1, x_ref.at[pl.ds(HALF + (QUART - a2), QUART), :], r2, q2)

        s1a.wait_recv()
        r1[pl.ds(f1, EIGHT), :] = (
            r1[pl.ds(f1, EIGHT), :] + x_ref[pl.ds(a1 + f1, EIGHT), :]
        )
        s2a = xchg(2, r1.at[pl.ds(f1, EIGHT), :], r3, q2)
        t1[:, :] = (
            r1[pl.ds(c1, EIGHT), :]
            + x_ref[pl.ds(o1, EIGHT), :]
            + resid_ref[pl.ds(o1, EIGHT), :]
        )

        s1b.wait_recv()
        r2[pl.ds(f2, EIGHT), :] = (
            r2[pl.ds(f2, EIGHT), :] + x_ref[pl.ds(HALF + a2 + f2, EIGHT), :]
        )
        s2b = xchg(3, r2.at[pl.ds(f2, EIGHT), :], r4, q1)
        t2[:, :] = (
            r2[pl.ds(c2, EIGHT), :]
            + x_ref[pl.ds(o2, EIGHT), :]
            + resid_ref[pl.ds(o2, EIGHT), :]
        )

        s2a.wait_recv()
        y1 = r3[:, :] + t1[:, :]
        inv1 = lax.rsqrt(jnp.mean(y1 * y1, axis=1, keepdims=True) + 1e-6)
        out_ref[pl.ds(o1, EIGHT), :] = y1 * inv1 * g_ref[0, :]
        s3a = xchg(4, out_ref.at[pl.ds(o1, EIGHT), :],
                   out_ref.at[pl.ds(o1, EIGHT), :], q2)

        s2b.wait_recv()
        y2 = r4[:, :] + t2[:, :]
        inv2 = lax.rsqrt(jnp.mean(y2 * y2, axis=1, keepdims=True) + 1e-6)
        out_ref[pl.ds(o2, EIGHT), :] = y2 * inv2 * g_ref[0, :]
        s3b = xchg(5, out_ref.at[pl.ds(o2, EIGHT), :],
                   out_ref.at[pl.ds(o2, EIGHT), :], q1)

        s3a.wait_recv()
        s4a = xchg(6, out_ref.at[pl.ds(a1, QUART), :],
                   out_ref.at[pl.ds(a1, QUART), :], q1)
        s3b.wait_recv()
        s4b = xchg(7, out_ref.at[pl.ds(HALF + a2, QUART), :],
                   out_ref.at[pl.ds(HALF + a2, QUART), :], q2)

        s4a.wait_recv()
        s4b.wait_recv()

        for s in (s1a, s1b, s2a, s2b, s3a, s3b, s4a, s4b):
            s.wait_send()

    return pl.pallas_call(
        body,
        out_shape=jax.ShapeDtypeStruct((M, D), jnp.float32),
        in_specs=[
            pl.BlockSpec(memory_space=pltpu.VMEM),
            pl.BlockSpec(memory_space=pltpu.VMEM),
            pl.BlockSpec(memory_space=pltpu.VMEM),
        ],
        out_specs=pl.BlockSpec(memory_space=pltpu.VMEM),
        scratch_shapes=[
            pltpu.VMEM((QUART, D), jnp.float32),
            pltpu.VMEM((QUART, D), jnp.float32),
            pltpu.VMEM((EIGHT, D), jnp.float32),
            pltpu.VMEM((EIGHT, D), jnp.float32),
            pltpu.VMEM((EIGHT, D), jnp.float32),
            pltpu.VMEM((EIGHT, D), jnp.float32),
            pltpu.SemaphoreType.DMA((8,)),
            pltpu.SemaphoreType.DMA((8,)),
        ],
        compiler_params=pltpu.CompilerParams(collective_id=0),
    )(x, resid, g)


# device time: 45821 ns/iter; 1.9144x vs baseline; 1.1054x over previous
import jax
import jax.numpy as jnp
from jax import lax
from jax.experimental import pallas as pl
from jax.experimental.pallas import tpu as pltpu

N_DEV = 4
M = 1024
D = 1024
HALF = M // 2
QUART = HALF // 2
EIGHT = HALF // 4


def kernel(partial, resid, gamma):
    x = partial.reshape(M, D)
    g = gamma.reshape(1, D)

    def body(x_ref, resid_ref, g_ref, out_ref, r1, r2, r3, r4, t1, t2,
             send_sems, recv_sems):
        p = lax.axis_index("i")
        q1 = p + 1 - 2 * (p % 2)
        q2 = 3 - p

        def xchg(idx, src, dst, partner):
            rdma = pltpu.make_async_remote_copy(
                src_ref=src, dst_ref=dst,
                send_sem=send_sems.at[idx], recv_sem=recv_sems.at[idx],
                device_id=(partner,), device_id_type=pl.DeviceIdType.MESH,
            )
            rdma.start()
            return rdma

        a1 = jnp.where((p == 0) | (p == 3), 0, QUART)
        c1 = jnp.where(p <= 1, 0, EIGHT)
        a2 = jnp.where(p <= 1, 0, QUART)
        c2 = jnp.where((p == 0) | (p == 2), 0, EIGHT)

        barrier_sem = pltpu.get_barrier_semaphore()
        for nbr in (q1, q2):
            pl.semaphore_signal(
                barrier_sem, inc=1,
                device_id=(nbr,), device_id_type=pl.DeviceIdType.MESH,
            )
        pl.semaphore_wait(barrier_sem, 2)

        o1 = a1 + c1
        o2 = HALF + a2 + c2
        f1 = EIGHT - c1
        f2 = EIGHT - c2
        SIX = EIGHT // 2

        s1a1 = xchg(0, x_ref.at[pl.ds((QUART - a1) + f1, EIGHT), :],
                    r1.at[pl.ds(f1, EIGHT), :], q1)
        s1a2 = xchg(1, x_ref.at[pl.ds((QUART - a1) + c1, EIGHT), :],
                    r1.at[pl.ds(c1, EIGHT), :], q1)
        s1b1 = xchg(2, x_ref.at[pl.ds(HALF + (QUART - a2) + c2, EIGHT), :],
                    r2.at[pl.ds(c2, EIGHT), :], q2)
        s1b2 = xchg(3, x_ref.at[pl.ds(HALF + (QUART - a2) + f2, EIGHT), :],
                    r2.at[pl.ds(f2, EIGHT), :], q2)

        s1a1.wait_recv()
        r1[pl.ds(f1, EIGHT), :] = (
            r1[pl.ds(f1, EIGHT), :] + x_ref[pl.ds(a1 + f1, EIGHT), :]
        )
        s2a1 = xchg(4, r1.at[pl.ds(f1, SIX), :], r3.at[pl.ds(0, SIX), :], q2)
        s2a2 = xchg(5, r1.at[pl.ds(f1 + SIX, SIX), :],
                    r3.at[pl.ds(SIX, SIX), :], q2)

        s1b1.wait_recv()
        r2[pl.ds(f2, EIGHT), :] = (
            r2[pl.ds(f2, EIGHT), :] + x_ref[pl.ds(HALF + a2 + f2, EIGHT), :]
        )
        s2b1 = xchg(6, r2.at[pl.ds(f2, SIX), :], r4.at[pl.ds(0, SIX), :], q1)
        s2b2 = xchg(7, r2.at[pl.ds(f2 + SIX, SIX), :],
                    r4.at[pl.ds(SIX, SIX), :], q1)

        s1a2.wait_recv()
        t1[:, :] = (
            r1[pl.ds(c1, EIGHT), :]
            + x_ref[pl.ds(o1, EIGHT), :]
            + resid_ref[pl.ds(o1, EIGHT), :]
        )
        s1b2.wait_recv()
        t2[:, :] = (
            r2[pl.ds(c2, EIGHT), :]
            + x_ref[pl.ds(o2, EIGHT), :]
            + resid_ref[pl.ds(o2, EIGHT), :]
        )

        def norm_store(rbuf, tbuf, j, dst_off):
            y = rbuf[pl.ds(j, SIX), :] + tbuf[pl.ds(j, SIX), :]
            inv = lax.rsqrt(jnp.mean(y * y, axis=1, keepdims=True) + 1e-6)
            out_ref[pl.ds(dst_off, SIX), :] = y * inv * g_ref[0, :]

        s2a1.wait_recv()
        norm_store(r3, t1, 0, o1)
        s3a1 = xchg(8, out_ref.at[pl.ds(o1, SIX), :],
                    out_ref.at[pl.ds(o1, SIX), :], q2)
        s2b1.wait_recv()
        norm_store(r4, t2, 0, o2)
        s3b1 = xchg(10, out_ref.at[pl.ds(o2, SIX), :],
                    out_ref.at[pl.ds(o2, SIX), :], q1)
        s2a2.wait_recv()
        norm_store(r3, t1, SIX, o1 + SIX)
        s3a2 = xchg(9, out_ref.at[pl.ds(o1 + SIX, SIX), :],
                    out_ref.at[pl.ds(o1 + SIX, SIX), :], q2)
        s2b2.wait_recv()
        norm_store(r4, t2, SIX, o2 + SIX)
        s3b2 = xchg(11, out_ref.at[pl.ds(o2 + SIX, SIX), :],
                    out_ref.at[pl.ds(o2 + SIX, SIX), :], q1)

        s4a1 = xchg(12, out_ref.at[pl.ds(o1, EIGHT), :],
                    out_ref.at[pl.ds(o1, EIGHT), :], q1)
        s4b1 = xchg(14, out_ref.at[pl.ds(o2, EIGHT), :],
                    out_ref.at[pl.ds(o2, EIGHT), :], q2)
        s3a1.wait_recv()
        s3a2.wait_recv()
        s4a2 = xchg(13, out_ref.at[pl.ds(a1 + f1, EIGHT), :],
                    out_ref.at[pl.ds(a1 + f1, EIGHT), :], q1)
        s3b1.wait_recv()
        s3b2.wait_recv()
        s4b2 = xchg(15, out_ref.at[pl.ds(HALF + a2 + f2, EIGHT), :],
                    out_ref.at[pl.ds(HALF + a2 + f2, EIGHT), :], q2)

        s4a1.wait_recv()
        s4a2.wait_recv()
        s4b1.wait_recv()
        s4b2.wait_recv()

        for s in (s1a1, s1a2, s1b1, s1b2, s2a1, s2a2, s2b1, s2b2,
                  s3a1, s3a2, s3b1, s3b2, s4a1, s4a2, s4b1, s4b2):
            s.wait_send()

    return pl.pallas_call(
        body,
        out_shape=jax.ShapeDtypeStruct((M, D), jnp.float32),
        in_specs=[
            pl.BlockSpec(memory_space=pltpu.VMEM),
            pl.BlockSpec(memory_space=pltpu.VMEM),
            pl.BlockSpec(memory_space=pltpu.VMEM),
        ],
        out_specs=pl.BlockSpec(memory_space=pltpu.VMEM),
        scratch_shapes=[
            pltpu.VMEM((QUART, D), jnp.float32),
            pltpu.VMEM((QUART, D), jnp.float32),
            pltpu.VMEM((EIGHT, D), jnp.float32),
            pltpu.VMEM((EIGHT, D), jnp.float32),
            pltpu.VMEM((EIGHT, D), jnp.float32),
            pltpu.VMEM((EIGHT, D), jnp.float32),
            pltpu.SemaphoreType.DMA((16,)),
            pltpu.SemaphoreType.DMA((16,)),
        ],
        compiler_params=pltpu.CompilerParams(collective_id=0),
    )(x, resid, g)
